# baseline (speedup 1.0000x reference)
import dataclasses
import functools

import jax
import jax.numpy as jnp
from jax import lax
from jax.experimental import pallas as pl
from jax.experimental.pallas import tpu as pltpu
from jax.experimental.pallas import tpu_sc as plsc

N = 4096
E = 131072
NFEAT = 716
F = 128
NC = 2
NS = 16
CHUNK = 128
NCHUNK = E // (NC * NS * CHUNK)
RPS = N // NS
NBUF = 4


def _spmm_body(sup_hbm, src_hbm, dst_hbm, w_hbm, out_hbm,
               idx_src, idx_dst, w_all, bufs, sem_g, sem_s, acc_sh):
    c = lax.axis_index("c")
    s = lax.axis_index("s")

    pltpu.sync_copy(src_hbm.at[c, s], idx_src)
    pltpu.sync_copy(dst_hbm.at[c, s], idx_dst)
    pltpu.sync_copy(w_hbm.at[c, s], w_all)

    rows0 = bufs[0]

    @pl.loop(0, CHUNK)
    def _zero(i):
        for j in range(F // 16):
            rows0[i, pl.ds(j * 16, 16)] = jnp.zeros((16,), jnp.float32)

    for r in range(RPS // CHUNK):
        pltpu.sync_copy(rows0, acc_sh.at[pl.ds(s * RPS + r * CHUNK, CHUNK)])
    plsc.subcore_barrier()

    def gather_start(g, k):
        pltpu.async_copy(sup_hbm.at[idx_src.at[g]], bufs[k], sem_g[k])

    def gather_wait(g, k):
        pltpu.make_async_copy(sup_hbm.at[idx_src.at[g]], bufs[k], sem_g[k]).wait()

    def scatter_start(g, k):
        pltpu.make_async_copy(bufs[k], acc_sh.at[idx_dst.at[g]], sem_s[k]).start(add=True)

    def scatter_wait(g, k):
        pltpu.make_async_copy(bufs[k], acc_sh.at[idx_dst.at[g]], sem_s[k]).wait()

    def scale(rows, g):
        gv = jnp.full((16,), g, jnp.int32)

        @plsc.parallel_loop(0, CHUNK, unroll=2)
        def _(e):
            wv = plsc.load_gather(w_all, [gv, jnp.full((16,), e, jnp.int32)])
            for j in range(F // 16):
                rows[e, pl.ds(j * 16, 16)] = rows[e, pl.ds(j * 16, 16)] * wv

    for k in range(NBUF - 1):
        gather_start(jnp.int32(k), k)

    @pl.loop(0, NCHUNK, step=NBUF)
    def _step(g4):
        for k in range(NBUF):
            g = g4 + k
            gather_wait(g, k)
            scale(bufs[k], g)
            scatter_start(g, k)
            kp = (k + NBUF - 1) % NBUF

            @pl.when(g >= 1)
            def _():
                scatter_wait(g, kp)

            @pl.when(g + (NBUF - 1) < NCHUNK)
            def _():
                gather_start(g + (NBUF - 1), kp)

    scatter_wait(jnp.int32(NCHUNK - 1), NBUF - 1)

    plsc.subcore_barrier()
    for r in range(RPS // CHUNK):
        base = s * RPS + r * CHUNK
        pltpu.sync_copy(acc_sh.at[pl.ds(base, CHUNK)], rows0)
        pltpu.sync_copy(rows0, out_hbm.at[c, pl.ds(base, CHUNK)])


def _spmm(sup, src, dst, w):
    mesh = plsc.VectorSubcoreMesh(core_axis_name="c", subcore_axis_name="s")
    cp = pltpu.CompilerParams()
    if "needs_layout_passes" in pltpu.CompilerParams.__dataclass_fields__:
        cp = dataclasses.replace(cp, needs_layout_passes=False)
    run = pl.kernel(
        _spmm_body,
        out_type=jax.ShapeDtypeStruct((2, N, F), jnp.float32),
        mesh=mesh,
        scratch_types=[
            pltpu.VMEM((NCHUNK, CHUNK), jnp.int32),
            pltpu.VMEM((NCHUNK, CHUNK), jnp.int32),
            pltpu.VMEM((NCHUNK, CHUNK), jnp.float32),
            [pltpu.VMEM((CHUNK, F), jnp.float32)] * NBUF,
            [pltpu.SemaphoreType.DMA] * NBUF,
            [pltpu.SemaphoreType.DMA] * NBUF,
            pltpu.VMEM_SHARED((N, F), jnp.float32),
        ],
        compiler_params=cp,
    )
    return run(sup, src, dst, w)


def _mm1_body(x_ref, w_ref, o_ref):
    o_ref[...] = jnp.dot(x_ref[...], w_ref[...],
                         preferred_element_type=jnp.float32)


def _matmul1(x, W):
    BM = 512
    K = x.shape[1]
    return pl.pallas_call(
        _mm1_body,
        grid=(N // BM,),
        in_specs=[
            pl.BlockSpec((BM, K), lambda m: (m, 0)),
            pl.BlockSpec((K, F), lambda m: (0, 0)),
        ],
        out_specs=pl.BlockSpec((BM, F), lambda m: (m, 0)),
        out_shape=jax.ShapeDtypeStruct((N, F), jnp.float32),
    )(x, W)


def _mm2_body(p_ref, b_ref, w_ref, o_ref):
    h = jnp.maximum(p_ref[0] + p_ref[1] + b_ref[...], 0.0)
    o_ref[...] = jnp.dot(h, w_ref[...], preferred_element_type=jnp.float32)


def _mm2(p, b1, W2):
    BM = 512
    return pl.pallas_call(
        _mm2_body,
        grid=(N // BM,),
        in_specs=[
            pl.BlockSpec((2, BM, F), lambda m: (0, m, 0)),
            pl.BlockSpec((1, F), lambda m: (0, 0)),
            pl.BlockSpec((F, F), lambda m: (0, 0)),
        ],
        out_specs=pl.BlockSpec((BM, F), lambda m: (m, 0)),
        out_shape=jax.ShapeDtypeStruct((N, F), jnp.float32),
    )(p, b1.reshape(1, F), W2)


def _bias_body(q_ref, b_ref, o_ref):
    o_ref[...] = q_ref[0] + q_ref[1] + b_ref[...]


def _bias_add(q, b2):
    BM = 512
    return pl.pallas_call(
        _bias_body,
        grid=(N // BM,),
        in_specs=[
            pl.BlockSpec((2, BM, F), lambda m: (0, m, 0)),
            pl.BlockSpec((1, F), lambda m: (0, 0)),
        ],
        out_specs=pl.BlockSpec((BM, F), lambda m: (m, 0)),
        out_shape=jax.ShapeDtypeStruct((N, F), jnp.float32),
    )(q, b2.reshape(1, F))


def _gram_body(a_ref, b_ref, o_ref):
    o_ref[...] = lax.dot_general(
        a_ref[...], b_ref[...], (((1,), (1,)), ((), ())),
        preferred_element_type=jnp.float32).astype(jnp.bfloat16)


def _gram(A):
    BM = 512
    return pl.pallas_call(
        _gram_body,
        grid=(N // BM, N // BM),
        in_specs=[
            pl.BlockSpec((BM, F), lambda i, j: (i, 0)),
            pl.BlockSpec((BM, F), lambda i, j: (j, 0)),
        ],
        out_specs=pl.BlockSpec((BM, BM), lambda i, j: (i, j)),
        out_shape=jax.ShapeDtypeStruct((N, N), jnp.bfloat16),
    )(A, A)


def _gram_sig_body(g_ref, a_ref, b_ref, o_ref):
    acc = lax.dot_general(a_ref[...], b_ref[...], (((1,), (1,)), ((), ())),
                          preferred_element_type=jnp.float32)
    o_ref[...] = jax.nn.sigmoid(acc + g_ref[...].astype(jnp.float32))


def _gram_sigmoid(G0, A):
    BM = 512
    return pl.pallas_call(
        _gram_sig_body,
        grid=(N // BM, N // BM),
        in_specs=[
            pl.BlockSpec((BM, BM), lambda i, j: (i, j)),
            pl.BlockSpec((BM, F), lambda i, j: (i, 0)),
            pl.BlockSpec((BM, F), lambda i, j: (j, 0)),
        ],
        out_specs=pl.BlockSpec((BM, BM), lambda i, j: (i, j)),
        out_shape=jax.ShapeDtypeStruct((N, N), jnp.float32),
    )(G0, A, A)


def kernel(x0, x1, edge_index0, edge_index1, edge_weight0, edge_weight1,
           W1_0, b1_0, W2_0, b2_0, W1_1, b1_1, W2_1, b2_1):
    esh = (NC, NS, NCHUNK, CHUNK)
    src0 = edge_index0[1].reshape(esh)
    dst0 = edge_index0[0].reshape(esh)
    w0 = edge_weight0.reshape(esh)
    src1 = edge_index1[1].reshape(esh)
    dst1 = edge_index1[0].reshape(esh)
    w1 = edge_weight1.reshape(esh)

    s0 = _matmul1(x0, W1_0)
    p0 = _spmm(s0, src0, dst0, w0)
    t0 = _mm2(p0, b1_0, W2_0)
    q0 = _spmm(t0, src0, dst0, w0)
    out0 = _bias_add(q0, b2_0)
    G0 = _gram(out0.astype(jnp.bfloat16))

    s1 = _matmul1(x1, W1_1)
    p1 = _spmm(s1, src1, dst1, w1)
    t1 = _mm2(p1, b1_1, W2_1)
    q1 = _spmm(t1, src1, dst1, w1)
    out1 = _bias_add(q1, b2_1)

    pred = _gram_sigmoid(G0, out1.astype(jnp.bfloat16))
    return (pred, out1)


# speedup vs baseline: 1.0744x; 1.0744x over previous
import dataclasses
import functools

import jax
import jax.numpy as jnp
from jax import lax
from jax.experimental import pallas as pl
from jax.experimental.pallas import tpu as pltpu
from jax.experimental.pallas import tpu_sc as plsc

N = 4096
E = 131072
NFEAT = 716
F = 128
NC = 2
NS = 16
CHUNK = 128
NCHUNK = E // (NS * CHUNK)
RPS = N // NS
NBUF = 4


def _spmm_body(sup_hbm, src_hbm, dst_hbm, w_hbm, out_hbm,
               idx_src, idx_dst, w_all, bufs, sem_g, sem_s, acc_sh):
    c = lax.axis_index("c")
    s = lax.axis_index("s")

    pltpu.sync_copy(src_hbm.at[c, s], idx_src)
    pltpu.sync_copy(dst_hbm.at[c, s], idx_dst)
    pltpu.sync_copy(w_hbm.at[c, s], w_all)

    rows0 = bufs[0]

    @pl.loop(0, CHUNK)
    def _zero(i):
        for j in range(F // 16):
            rows0[i, pl.ds(j * 16, 16)] = jnp.zeros((16,), jnp.float32)

    for r in range(RPS // CHUNK):
        pltpu.sync_copy(rows0, acc_sh.at[pl.ds(s * RPS + r * CHUNK, CHUNK)])
    plsc.subcore_barrier()

    def gather_start(g, k):
        pltpu.async_copy(sup_hbm.at[idx_src.at[g]], bufs[k], sem_g[k])

    def gather_wait(g, k):
        pltpu.make_async_copy(sup_hbm.at[idx_src.at[g]], bufs[k], sem_g[k]).wait()

    def scatter_start(g, k):
        pltpu.make_async_copy(bufs[k], acc_sh.at[idx_dst.at[g]], sem_s[k]).start(add=True)

    def scatter_wait(g, k):
        pltpu.make_async_copy(bufs[k], acc_sh.at[idx_dst.at[g]], sem_s[k]).wait()

    def scale(rows, g):
        gv = jnp.full((16,), g, jnp.int32)

        @plsc.parallel_loop(0, CHUNK, unroll=2)
        def _(e):
            wv = plsc.load_gather(w_all, [gv, jnp.full((16,), e, jnp.int32)])
            for j in range(F // 16):
                rows[e, pl.ds(j * 16, 16)] = rows[e, pl.ds(j * 16, 16)] * wv

    for k in range(NBUF - 1):
        gather_start(jnp.int32(k), k)

    @pl.loop(0, NCHUNK, step=NBUF)
    def _step(g4):
        for k in range(NBUF):
            g = g4 + k
            gather_wait(g, k)
            scale(bufs[k], g)
            scatter_start(g, k)
            kp = (k + NBUF - 1) % NBUF

            @pl.when(g >= 1)
            def _():
                scatter_wait(g, kp)

            @pl.when(g + (NBUF - 1) < NCHUNK)
            def _():
                gather_start(g + (NBUF - 1), kp)

    scatter_wait(jnp.int32(NCHUNK - 1), NBUF - 1)

    plsc.subcore_barrier()
    pltpu.sync_copy(acc_sh.at[pl.ds(s * RPS, RPS)],
                    out_hbm.at[c, pl.ds(s * RPS, RPS)])


def _spmm(sup, src, dst, w):
    mesh = plsc.VectorSubcoreMesh(core_axis_name="c", subcore_axis_name="s")
    cp = pltpu.CompilerParams()
    if "needs_layout_passes" in pltpu.CompilerParams.__dataclass_fields__:
        cp = dataclasses.replace(cp, needs_layout_passes=False)
    run = pl.kernel(
        _spmm_body,
        out_type=jax.ShapeDtypeStruct((2, N, F), jnp.float32),
        mesh=mesh,
        scratch_types=[
            pltpu.VMEM((NCHUNK, CHUNK), jnp.int32),
            pltpu.VMEM((NCHUNK, CHUNK), jnp.int32),
            pltpu.VMEM((NCHUNK, CHUNK), jnp.float32),
            [pltpu.VMEM((CHUNK, F), jnp.float32)] * NBUF,
            [pltpu.SemaphoreType.DMA] * NBUF,
            [pltpu.SemaphoreType.DMA] * NBUF,
            pltpu.VMEM_SHARED((N, F), jnp.float32),
        ],
        compiler_params=cp,
    )
    return run(sup, src, dst, w)


def _mm1_body(x_ref, w_ref, o_ref):
    o_ref[0] = jnp.dot(x_ref[0], w_ref[0], preferred_element_type=jnp.float32)


def _matmul1(xs, Ws):
    BM = 512
    K = xs.shape[2]
    return pl.pallas_call(
        _mm1_body,
        grid=(2, N // BM),
        in_specs=[
            pl.BlockSpec((1, BM, K), lambda g, m: (g, m, 0)),
            pl.BlockSpec((1, K, F), lambda g, m: (g, 0, 0)),
        ],
        out_specs=pl.BlockSpec((1, BM, F), lambda g, m: (g, m, 0)),
        out_shape=jax.ShapeDtypeStruct((2, N, F), jnp.float32),
    )(xs, Ws)


def _mm2_body(p_ref, b_ref, w_ref, o_ref):
    h = jnp.maximum(p_ref[0] + b_ref[pl.program_id(0)][None, :], 0.0)
    o_ref[0] = jnp.dot(h, w_ref[0], preferred_element_type=jnp.float32)


def _mm2(p, b1s, Ws2):
    BM = 512
    return pl.pallas_call(
        _mm2_body,
        grid=(2, N // BM),
        in_specs=[
            pl.BlockSpec((1, BM, F), lambda g, m: (g, m, 0)),
            pl.BlockSpec((2, F), lambda g, m: (0, 0)),
            pl.BlockSpec((1, F, F), lambda g, m: (g, 0, 0)),
        ],
        out_specs=pl.BlockSpec((1, BM, F), lambda g, m: (g, m, 0)),
        out_shape=jax.ShapeDtypeStruct((2, N, F), jnp.float32),
    )(p, b1s, Ws2)


def _bias_body(q_ref, b_ref, o_ref):
    o_ref[0] = q_ref[0] + b_ref[pl.program_id(0)][None, :]


def _bias_add(q, b2s):
    BM = 512
    return pl.pallas_call(
        _bias_body,
        grid=(2, N // BM),
        in_specs=[
            pl.BlockSpec((1, BM, F), lambda g, m: (g, m, 0)),
            pl.BlockSpec((2, F), lambda g, m: (0, 0)),
        ],
        out_specs=pl.BlockSpec((1, BM, F), lambda g, m: (g, m, 0)),
        out_shape=jax.ShapeDtypeStruct((2, N, F), jnp.float32),
    )(q, b2s)


def _pred_body(a_ref, b_ref, o_ref):
    dn = (((1,), (1,)), ((), ()))
    acc = lax.dot_general(a_ref[0], b_ref[0], dn,
                          preferred_element_type=jnp.float32)
    acc += lax.dot_general(a_ref[1], b_ref[1], dn,
                           preferred_element_type=jnp.float32)
    o_ref[...] = jax.nn.sigmoid(acc)


def _pred(out_bf):
    BM = 512
    return pl.pallas_call(
        _pred_body,
        grid=(N // BM, N // BM),
        in_specs=[
            pl.BlockSpec((2, BM, F), lambda i, j: (0, i, 0)),
            pl.BlockSpec((2, BM, F), lambda i, j: (0, j, 0)),
        ],
        out_specs=pl.BlockSpec((BM, BM), lambda i, j: (i, j)),
        out_shape=jax.ShapeDtypeStruct((N, N), jnp.float32),
    )(out_bf, out_bf)


def kernel(x0, x1, edge_index0, edge_index1, edge_weight0, edge_weight1,
           W1_0, b1_0, W2_0, b2_0, W1_1, b1_1, W2_1, b2_1):
    xs = jnp.stack([x0, x1])
    Ws1 = jnp.stack([W1_0, W1_1])
    b1s = jnp.stack([b1_0, b1_1])
    Ws2 = jnp.stack([W2_0, W2_1])
    b2s = jnp.stack([b2_0, b2_1])
    esh = (NC, NS, NCHUNK, CHUNK)
    src = jnp.stack([edge_index0[1], edge_index1[1] + N]).reshape(esh)
    dst = jnp.stack([edge_index0[0], edge_index1[0]]).reshape(esh)
    w = jnp.stack([edge_weight0, edge_weight1]).reshape(esh)

    support = _matmul1(xs, Ws1)
    p = _spmm(support.reshape(2 * N, F), src, dst, w)
    support2 = _mm2(p, b1s, Ws2)
    q = _spmm(support2.reshape(2 * N, F), src, dst, w)
    out = _bias_add(q, b2s)
    pred = _pred(out.astype(jnp.bfloat16))
    return (pred, out[1])


# speedup vs baseline: 1.1774x; 1.0958x over previous
import dataclasses
import functools

import jax
import jax.numpy as jnp
from jax import lax
from jax.experimental import pallas as pl
from jax.experimental.pallas import tpu as pltpu
from jax.experimental.pallas import tpu_sc as plsc

N = 4096
E = 131072
NFEAT = 716
F = 128
NC = 2
NS = 16
CHUNK = 128
NCHUNK = E // (NS * CHUNK)
RPS = N // NS
NBUF = 4


def _spmm_body(sup_hbm, src_hbm, dst_hbm, w_hbm, out_hbm,
               idx_src, idx_dst, w_all, bufs, sem_g, sem_s, acc_sh):
    c = lax.axis_index("c")
    s = lax.axis_index("s")

    pltpu.sync_copy(src_hbm.at[c, s], idx_src)
    pltpu.sync_copy(dst_hbm.at[c, s], idx_dst)
    pltpu.sync_copy(w_hbm.at[c, s], w_all)

    rows0 = bufs[0]

    @pl.loop(0, CHUNK)
    def _zero(i):
        for j in range(F // 16):
            rows0[i, pl.ds(j * 16, 16)] = jnp.zeros((16,), jnp.float32)

    for r in range(RPS // CHUNK):
        pltpu.sync_copy(rows0, acc_sh.at[pl.ds(s * RPS + r * CHUNK, CHUNK)])
    plsc.subcore_barrier()

    def gather_start(g, k):
        pltpu.async_copy(sup_hbm.at[idx_src.at[g]], bufs[k], sem_g[k])

    def gather_wait(g, k):
        pltpu.make_async_copy(sup_hbm.at[idx_src.at[g]], bufs[k], sem_g[k]).wait()

    def scatter_start(g, k):
        pltpu.make_async_copy(bufs[k], acc_sh.at[idx_dst.at[g]], sem_s[k]).start(add=True)

    def scatter_wait(g, k):
        pltpu.make_async_copy(bufs[k], acc_sh.at[idx_dst.at[g]], sem_s[k]).wait()

    def scale(rows, g):
        gv = jnp.full((16,), g, jnp.int32)

        @plsc.parallel_loop(0, CHUNK, unroll=2)
        def _(e):
            wv = plsc.load_gather(w_all, [gv, jnp.full((16,), e, jnp.int32)])
            for j in range(F // 16):
                rows[e, pl.ds(j * 16, 16)] = rows[e, pl.ds(j * 16, 16)] * wv

    for k in range(NBUF - 1):
        gather_start(jnp.int32(k), k)

    @pl.loop(0, NCHUNK, step=NBUF)
    def _step(g4):
        for k in range(NBUF):
            g = g4 + k
            gather_wait(g, k)
            scale(bufs[k], g)
            scatter_start(g, k)
            kp = (k + NBUF - 1) % NBUF

            @pl.when(g >= 1)
            def _():
                scatter_wait(g, kp)

            @pl.when(g + (NBUF - 1) < NCHUNK)
            def _():
                gather_start(g + (NBUF - 1), kp)

    scatter_wait(jnp.int32(NCHUNK - 1), NBUF - 1)

    plsc.subcore_barrier()
    pltpu.sync_copy(acc_sh.at[pl.ds(s * RPS, RPS)],
                    out_hbm.at[c, pl.ds(s * RPS, RPS)])


def _spmm(sup, src, dst, w):
    mesh = plsc.VectorSubcoreMesh(core_axis_name="c", subcore_axis_name="s")
    cp = pltpu.CompilerParams()
    if "needs_layout_passes" in pltpu.CompilerParams.__dataclass_fields__:
        cp = dataclasses.replace(cp, needs_layout_passes=False)
    run = pl.kernel(
        _spmm_body,
        out_type=jax.ShapeDtypeStruct((2, N, F), jnp.float32),
        mesh=mesh,
        scratch_types=[
            pltpu.VMEM((NCHUNK, CHUNK), jnp.int32),
            pltpu.VMEM((NCHUNK, CHUNK), jnp.int32),
            pltpu.VMEM((NCHUNK, CHUNK), jnp.float32),
            [pltpu.VMEM((CHUNK, F), jnp.float32)] * NBUF,
            [pltpu.SemaphoreType.DMA] * NBUF,
            [pltpu.SemaphoreType.DMA] * NBUF,
            pltpu.VMEM_SHARED((N, F), jnp.float32),
        ],
        compiler_params=cp,
    )
    return run(sup, src, dst, w)


def _mm1_body(x0_ref, x1_ref, w0_ref, w1_ref, o_ref):
    o_ref[0] = jnp.dot(x0_ref[...], w0_ref[...],
                       preferred_element_type=jnp.float32)
    o_ref[1] = jnp.dot(x1_ref[...], w1_ref[...],
                       preferred_element_type=jnp.float32)


def _matmul1(x0, x1, W1_0, W1_1):
    BM = 512
    K = x0.shape[1]
    return pl.pallas_call(
        _mm1_body,
        grid=(N // BM,),
        in_specs=[
            pl.BlockSpec((BM, K), lambda m: (m, 0)),
            pl.BlockSpec((BM, K), lambda m: (m, 0)),
            pl.BlockSpec((K, F), lambda m: (0, 0)),
            pl.BlockSpec((K, F), lambda m: (0, 0)),
        ],
        out_specs=pl.BlockSpec((2, BM, F), lambda m: (0, m, 0)),
        out_shape=jax.ShapeDtypeStruct((2, N, F), jnp.float32),
    )(x0, x1, W1_0, W1_1)


def _mm2_body(p_ref, b0_ref, b1_ref, w0_ref, w1_ref, o_ref):
    h0 = jnp.maximum(p_ref[0] + b0_ref[...], 0.0)
    h1 = jnp.maximum(p_ref[1] + b1_ref[...], 0.0)
    o_ref[0] = jnp.dot(h0, w0_ref[...], preferred_element_type=jnp.float32)
    o_ref[1] = jnp.dot(h1, w1_ref[...], preferred_element_type=jnp.float32)


def _mm2(p, b1_0, b1_1, W2_0, W2_1):
    BM = 512
    return pl.pallas_call(
        _mm2_body,
        grid=(N // BM,),
        in_specs=[
            pl.BlockSpec((2, BM, F), lambda m: (0, m, 0)),
            pl.BlockSpec((1, F), lambda m: (0, 0)),
            pl.BlockSpec((1, F), lambda m: (0, 0)),
            pl.BlockSpec((F, F), lambda m: (0, 0)),
            pl.BlockSpec((F, F), lambda m: (0, 0)),
        ],
        out_specs=pl.BlockSpec((2, BM, F), lambda m: (0, m, 0)),
        out_shape=jax.ShapeDtypeStruct((2, N, F), jnp.float32),
    )(p, b1_0.reshape(1, F), b1_1.reshape(1, F), W2_0, W2_1)


def _bias_body(q_ref, b0_ref, b1_ref, obf_ref, o1_ref):
    v0 = q_ref[0] + b0_ref[...]
    v1 = q_ref[1] + b1_ref[...]
    o1_ref[...] = v1
    obf_ref[0] = v0.astype(jnp.bfloat16)
    obf_ref[1] = v1.astype(jnp.bfloat16)


def _bias_add(q, b2_0, b2_1):
    BM = 512
    return pl.pallas_call(
        _bias_body,
        grid=(N // BM,),
        in_specs=[
            pl.BlockSpec((2, BM, F), lambda m: (0, m, 0)),
            pl.BlockSpec((1, F), lambda m: (0, 0)),
            pl.BlockSpec((1, F), lambda m: (0, 0)),
        ],
        out_specs=[
            pl.BlockSpec((2, BM, F), lambda m: (0, m, 0)),
            pl.BlockSpec((BM, F), lambda m: (m, 0)),
        ],
        out_shape=[
            jax.ShapeDtypeStruct((2, N, F), jnp.bfloat16),
            jax.ShapeDtypeStruct((N, F), jnp.float32),
        ],
    )(q, b2_0.reshape(1, F), b2_1.reshape(1, F))


def _pred_body(a_ref, b_ref, o_ref):
    dn = (((1,), (1,)), ((), ()))
    acc = lax.dot_general(a_ref[0], b_ref[0], dn,
                          preferred_element_type=jnp.float32)
    acc += lax.dot_general(a_ref[1], b_ref[1], dn,
                           preferred_element_type=jnp.float32)
    o_ref[...] = jax.nn.sigmoid(acc)


def _pred(out_bf):
    BM = 512
    return pl.pallas_call(
        _pred_body,
        grid=(N // BM, N // BM),
        in_specs=[
            pl.BlockSpec((2, BM, F), lambda i, j: (0, i, 0)),
            pl.BlockSpec((2, BM, F), lambda i, j: (0, j, 0)),
        ],
        out_specs=pl.BlockSpec((BM, BM), lambda i, j: (i, j)),
        out_shape=jax.ShapeDtypeStruct((N, N), jnp.float32),
    )(out_bf, out_bf)


def kernel(x0, x1, edge_index0, edge_index1, edge_weight0, edge_weight1,
           W1_0, b1_0, W2_0, b2_0, W1_1, b1_1, W2_1, b2_1):
    esh = (NC, NS, NCHUNK, CHUNK)
    src = jnp.stack([edge_index0[1], edge_index1[1] + N]).reshape(esh)
    dst = jnp.stack([edge_index0[0], edge_index1[0]]).reshape(esh)
    w = jnp.stack([edge_weight0, edge_weight1]).reshape(esh)

    support = _matmul1(x0, x1, W1_0, W1_1)
    p = _spmm(support.reshape(2 * N, F), src, dst, w)
    support2 = _mm2(p, b1_0, b1_1, W2_0, W2_1)
    q = _spmm(support2.reshape(2 * N, F), src, dst, w)
    out_bf, out1 = _bias_add(q, b2_0, b2_1)
    pred = _pred(out_bf)
    return (pred, out1)
